# baseline (speedup 1.0000x reference)
import jax
import jax.numpy as jnp
from jax.experimental import pallas as pl
from jax.experimental.pallas import tpu as pltpu


def _crelu(x, weight, gamma, beta, running_mean, running_var,
           stride=4, padding=3, eps=1e-5):
    N, C, H, W = x.shape
    Cout, Cin, KH, KW = weight.shape
    s = int(stride)
    p = int(padding)
    assert s == 4 and p == 3 and Cin == C and KH == 7 and KW == 7
    OH = (H + 2 * p - KH) // s + 1
    OW = (W + 2 * p - KW) // s + 1
    KHs = 2
    KWs = 2
    CE = s * s * C

    T = next(t for t in (32, 16, 8, 4, 2, 1) if OH % t == 0)
    R = OH // T
    HB = H // R
    Hs = OH + KHs
    WSP = (OW + KWs + 127) // 128 * 128
    L = (T + KHs) * WSP
    TQ = T * WSP
    NP = -(-OW // 128)

    scale = gamma.astype(jnp.float32) * jax.lax.rsqrt(
        running_var.astype(jnp.float32) + eps)
    bias = (beta.astype(jnp.float32)
            - running_mean.astype(jnp.float32) * scale).reshape(Cout, 1)
    w_f = weight.astype(jnp.float32) * scale[:, None, None, None]

    w_pad = jnp.pad(w_f, ((0, 0), (0, 0), (0, KHs * s - KH), (0, KWs * s - KW)))
    wg = w_pad.reshape(Cout, C, KHs, s, KWs, s)
    wg = wg.transpose(2, 4, 0, 3, 5, 1)
    w_all = wg.reshape(KHs * KWs * Cout, CE).astype(jnp.bfloat16)

    M = KHs * KWs * Cout

    def body(x_ref, w_ref, b_ref, o_ref, xs3, xt_s, xw_s):
        k = pl.program_id(1)

        @pl.when(k == 0)
        def _zero():
            xs3[...] = jnp.zeros((CE, Hs, WSP), jnp.float32)

        @pl.when(k < R)
        def _build():
            rows = x_ref[...]
            xt_s[...] = jnp.swapaxes(rows, 1, 2)
            for t in range(s):
                wp0 = 0 if t == 3 else 1
                cs0 = 4 * wp0 + t - 3
                colsT = xt_s[:, cs0:cs0 + 4 * OW:4, :]
                cols = jnp.swapaxes(colsT, 1, 2)
                for i in range(NP):
                    pw = min(128, OW - 128 * i)
                    xw_s[:, t, i, :, 0:pw] = cols[:, :, 128 * i:128 * i + pw]
            for kk in range(R):
                @pl.when(k == kk)
                def _store(kk=kk):
                    for rt in range(s):
                        hs0 = 0 if rt == 3 else 1
                        rs0 = 4 * hs0 + rt - 3
                        for t in range(s):
                            wp0 = 0 if t == 3 else 1
                            ce0 = (rt * s + t) * C
                            part = xw_s[:, t, :, rs0:rs0 + 4 * T:4, :]
                            for i in range(NP):
                                pw = min(128, OW - 128 * i)
                                xs3[ce0:ce0 + C,
                                    hs0 + T * kk:hs0 + T * kk + T,
                                    wp0 + 128 * i:wp0 + 128 * i + pw] = (
                                    part[:, i, :, 0:pw])

        @pl.when(k > 0)
        def _compute():
            kk = k - 1
            xf = xs3[:, pl.ds(T * kk, T + KHs), :].astype(jnp.bfloat16)
            xf = xf.reshape(CE, L)
            prod = jnp.dot(w_ref[...], xf,
                           preferred_element_type=jnp.float32)
            acc = (prod[0 * Cout:1 * Cout, 0:TQ]
                   + prod[1 * Cout:2 * Cout, 1:TQ + 1]
                   + prod[2 * Cout:3 * Cout, WSP:WSP + TQ]
                   + prod[3 * Cout:4 * Cout, WSP + 1:WSP + TQ + 1])
            bn = acc + b_ref[...]
            pos = jnp.maximum(bn, 0.0)
            neg = pos - bn
            ob = jnp.concatenate([pos, neg], axis=0).astype(o_ref.dtype)
            o_ref[...] = ob.reshape(2 * Cout, T, WSP)[:, :, :OW]

    cost = pl.CostEstimate(
        flops=2 * N * R * M * CE * L,
        transcendentals=0,
        bytes_accessed=(x.size * 4 + w_all.size * 2 + bias.size * 4
                        + N * 2 * Cout * OH * OW * 2),
    )

    out = pl.pallas_call(
        body,
        out_shape=jax.ShapeDtypeStruct((N, 2 * Cout, OH, OW), jnp.bfloat16),
        grid=(N, R + 1),
        in_specs=[
            pl.BlockSpec((None, C, HB, W),
                         lambda n, r: (n, 0, jnp.minimum(r, R - 1), 0)),
            pl.BlockSpec((M, CE), lambda n, r: (0, 0)),
            pl.BlockSpec((Cout, 1), lambda n, r: (0, 0)),
        ],
        out_specs=pl.BlockSpec((None, 2 * Cout, T, OW),
                               lambda n, r: (n, 0, jnp.maximum(r - 1, 0), 0)),
        scratch_shapes=[pltpu.VMEM((CE, Hs, WSP), jnp.float32),
                        pltpu.VMEM((C, W, HB), jnp.float32),
                        pltpu.VMEM((C, s, NP, HB, 128), jnp.float32)],
        compiler_params=pltpu.CompilerParams(
            dimension_semantics=("parallel", "arbitrary"),
            vmem_limit_bytes=100 * 1024 * 1024),
        cost_estimate=cost,
    )(x, w_all, bias)
    return out


@jax.jit
def kernel(x, weight, gamma, beta, running_mean, running_var):
    return _crelu(x, weight, gamma, beta, running_mean, running_var,
                  stride=4, padding=3)


# speedup vs baseline: 1.0114x; 1.0114x over previous
import jax
import jax.numpy as jnp
from jax.experimental import pallas as pl
from jax.experimental.pallas import tpu as pltpu


def _crelu(x, weight, gamma, beta, running_mean, running_var,
           stride=4, padding=3, eps=1e-5):
    N, C, H, W = x.shape
    Cout, Cin, KH, KW = weight.shape
    s = int(stride)
    p = int(padding)
    assert s == 4 and p == 3 and Cin == C and KH == 7 and KW == 7
    OH = (H + 2 * p - KH) // s + 1
    OW = (W + 2 * p - KW) // s + 1
    KHs = 2
    KWs = 2
    CE = s * s * C

    T = next(t for t in (32, 16, 8, 4, 2, 1) if OH % t == 0)
    R = OH // T
    HB = H // R
    Hs = OH + KHs
    WSP = (OW + KWs + 127) // 128 * 128
    L = (T + KHs) * WSP
    TQ = T * WSP
    NP = -(-OW // 128)

    scale = gamma.astype(jnp.float32) * jax.lax.rsqrt(
        running_var.astype(jnp.float32) + eps)
    bias = (beta.astype(jnp.float32)
            - running_mean.astype(jnp.float32) * scale).reshape(Cout, 1)
    w_f = weight.astype(jnp.float32) * scale[:, None, None, None]

    w_pad = jnp.pad(w_f, ((0, 0), (0, 0), (0, KHs * s - KH), (0, KWs * s - KW)))
    wg = w_pad.reshape(Cout, C, KHs, s, KWs, s)
    wg = wg.transpose(2, 4, 0, 3, 5, 1)
    w_all = wg.reshape(KHs * KWs * Cout, CE).astype(jnp.bfloat16)

    M = KHs * KWs * Cout

    def body(x_ref, w_ref, b_ref, o_ref, xs3, xt_s, xw_s):
        k = pl.program_id(1)

        @pl.when(k == 0)
        def _zero():
            z_r = jnp.zeros((CE, 1, WSP), jnp.bfloat16)
            xs3[:, 0:1, :] = z_r
            xs3[:, OH:OH + KHs, :] = jnp.zeros((CE, KHs, WSP), jnp.bfloat16)
            z_c = jnp.zeros((C * (s - 1), Hs, 1), jnp.bfloat16)
            z_c3 = jnp.zeros((C, Hs, 1), jnp.bfloat16)
            for rt in range(s):
                xs3[rt * s * C:(rt * s + 3) * C, :, 0:1] = z_c
                xs3[(rt * s + 3) * C:(rt + 1) * s * C, :, OW:OW + 1] = z_c3

        @pl.when(k < R)
        def _build():
            rows = x_ref[...]
            xt_s[...] = jnp.swapaxes(rows, 1, 2)
            for t in range(s):
                wp0 = 0 if t == 3 else 1
                cs0 = 4 * wp0 + t - 3
                colsT = xt_s[:, cs0:cs0 + 4 * OW:4, :]
                cols = jnp.swapaxes(colsT, 1, 2)
                for i in range(NP):
                    pw = min(128, OW - 128 * i)
                    xw_s[:, t, i, :, 0:pw] = cols[:, :, 128 * i:128 * i + pw]
            for kk in range(R):
                @pl.when(k == kk)
                def _store(kk=kk):
                    for rt in range(s):
                        hs0 = 0 if rt == 3 else 1
                        rs0 = 4 * hs0 + rt - 3
                        for t in range(s):
                            wp0 = 0 if t == 3 else 1
                            ce0 = (rt * s + t) * C
                            part = xw_s[:, t, :, rs0:rs0 + 4 * T:4,
                                        :].astype(jnp.bfloat16)
                            for i in range(NP):
                                pw = min(128, OW - 128 * i)
                                xs3[ce0:ce0 + C,
                                    hs0 + T * kk:hs0 + T * kk + T,
                                    wp0 + 128 * i:wp0 + 128 * i + pw] = (
                                    part[:, i, :, 0:pw])

        @pl.when(k > 0)
        def _compute():
            kk = k - 1
            xf = xs3[:, pl.ds(T * kk, T + KHs), :].reshape(CE, L)
            prod = jnp.dot(w_ref[...], xf,
                           preferred_element_type=jnp.float32)
            acc = (prod[0 * Cout:1 * Cout, 0:TQ]
                   + prod[1 * Cout:2 * Cout, 1:TQ + 1]
                   + prod[2 * Cout:3 * Cout, WSP:WSP + TQ]
                   + prod[3 * Cout:4 * Cout, WSP + 1:WSP + TQ + 1])
            bn = acc + b_ref[...]
            pos = jnp.maximum(bn, 0.0)
            neg = pos - bn
            ob = jnp.concatenate([pos, neg], axis=0).astype(o_ref.dtype)
            o_ref[...] = ob.reshape(2 * Cout, T, WSP)[:, :, :OW]

    cost = pl.CostEstimate(
        flops=2 * N * R * M * CE * L,
        transcendentals=0,
        bytes_accessed=(x.size * 4 + w_all.size * 2 + bias.size * 4
                        + N * 2 * Cout * OH * OW * 2),
    )

    out = pl.pallas_call(
        body,
        out_shape=jax.ShapeDtypeStruct((N, 2 * Cout, OH, OW), jnp.bfloat16),
        grid=(N, R + 1),
        in_specs=[
            pl.BlockSpec((None, C, HB, W),
                         lambda n, r: (n, 0, jnp.minimum(r, R - 1), 0)),
            pl.BlockSpec((M, CE), lambda n, r: (0, 0)),
            pl.BlockSpec((Cout, 1), lambda n, r: (0, 0)),
        ],
        out_specs=pl.BlockSpec((None, 2 * Cout, T, OW),
                               lambda n, r: (n, 0, jnp.maximum(r - 1, 0), 0)),
        scratch_shapes=[pltpu.VMEM((CE, Hs, WSP), jnp.bfloat16),
                        pltpu.VMEM((C, W, HB), jnp.float32),
                        pltpu.VMEM((C, s, NP, HB, 128), jnp.float32)],
        compiler_params=pltpu.CompilerParams(
            dimension_semantics=("parallel", "arbitrary"),
            vmem_limit_bytes=100 * 1024 * 1024),
        cost_estimate=cost,
    )(x, w_all, bias)
    return out


@jax.jit
def kernel(x, weight, gamma, beta, running_mean, running_var):
    return _crelu(x, weight, gamma, beta, running_mean, running_var,
                  stride=4, padding=3)


# speedup vs baseline: 1.1014x; 1.0890x over previous
import jax
import jax.numpy as jnp
from jax.experimental import pallas as pl
from jax.experimental.pallas import tpu as pltpu


def _crelu(x, weight, gamma, beta, running_mean, running_var,
           stride=4, padding=3, eps=1e-5):
    N, C, H, W = x.shape
    Cout, Cin, KH, KW = weight.shape
    s = int(stride)
    p = int(padding)
    assert s == 4 and p == 3 and Cin == C and KH == 7 and KW == 7
    OH = (H + 2 * p - KH) // s + 1
    OW = (W + 2 * p - KW) // s + 1
    KHs = 2
    KWs = 2
    CE = s * s * C

    T = next(t for t in (64, 32, 16, 8, 4, 2, 1) if OH % t == 0)
    R = OH // T
    HB = H // R
    Hs = OH + KHs
    WSP = (OW + KWs + 127) // 128 * 128
    L = (T + KHs) * WSP
    TQ = T * WSP
    NP = -(-OW // 128)
    CHK = min(128, HB)

    scale = gamma.astype(jnp.float32) * jax.lax.rsqrt(
        running_var.astype(jnp.float32) + eps)
    bias = (beta.astype(jnp.float32)
            - running_mean.astype(jnp.float32) * scale).reshape(Cout, 1)
    w_f = weight.astype(jnp.float32) * scale[:, None, None, None]

    w_pad = jnp.pad(w_f, ((0, 0), (0, 0), (0, KHs * s - KH), (0, KWs * s - KW)))
    wg = w_pad.reshape(Cout, C, KHs, s, KWs, s)
    wg = wg.transpose(2, 4, 0, 3, 5, 1)
    w_all = wg.reshape(KHs * KWs * Cout, CE).astype(jnp.bfloat16)

    M = KHs * KWs * Cout

    def body(x_ref, w_ref, b_ref, o_ref, xs3, xt_s, xw_s):
        k = pl.program_id(1)

        @pl.when(k == 0)
        def _zero():
            z_r = jnp.zeros((CE, 1, WSP), jnp.bfloat16)
            xs3[:, 0:1, :] = z_r
            xs3[:, OH:OH + KHs, :] = jnp.zeros((CE, KHs, WSP), jnp.bfloat16)
            z_c = jnp.zeros((C * (s - 1), Hs, 1), jnp.bfloat16)
            z_c3 = jnp.zeros((C, Hs, 1), jnp.bfloat16)
            for rt in range(s):
                xs3[rt * s * C:(rt * s + 3) * C, :, 0:1] = z_c
                xs3[(rt * s + 3) * C:(rt + 1) * s * C, :, OW:OW + 1] = z_c3

        @pl.when(k < R)
        def _build():
            for kc in range(HB // CHK):
                rows = x_ref[:, kc * CHK:(kc + 1) * CHK, :]
                xt_s[...] = jnp.swapaxes(rows, 1, 2)
                for t in range(s):
                    wp0 = 0 if t == 3 else 1
                    cs0 = 4 * wp0 + t - 3
                    colsT = xt_s[:, cs0:cs0 + 4 * OW:4, :]
                    cols = jnp.swapaxes(colsT, 1, 2)
                    for i in range(NP):
                        pw = min(128, OW - 128 * i)
                        xw_s[:, t, i, kc * CHK:(kc + 1) * CHK, 0:pw] = (
                            cols[:, :, 128 * i:128 * i + pw])
            for kk in range(R):
                @pl.when(k == kk)
                def _store(kk=kk):
                    for rt in range(s):
                        hs0 = 0 if rt == 3 else 1
                        rs0 = 4 * hs0 + rt - 3
                        for t in range(s):
                            wp0 = 0 if t == 3 else 1
                            ce0 = (rt * s + t) * C
                            part = xw_s[:, t, :, rs0:rs0 + 4 * T:4,
                                        :].astype(jnp.bfloat16)
                            for i in range(NP):
                                pw = min(128, OW - 128 * i)
                                xs3[ce0:ce0 + C,
                                    hs0 + T * kk:hs0 + T * kk + T,
                                    wp0 + 128 * i:wp0 + 128 * i + pw] = (
                                    part[:, i, :, 0:pw])

        @pl.when(k > 0)
        def _compute():
            kk = k - 1
            xf = xs3[:, pl.ds(T * kk, T + KHs), :].reshape(CE, L)
            prod = jnp.dot(w_ref[...], xf,
                           preferred_element_type=jnp.float32)
            acc = (prod[0 * Cout:1 * Cout, 0:TQ]
                   + prod[1 * Cout:2 * Cout, 1:TQ + 1]
                   + prod[2 * Cout:3 * Cout, WSP:WSP + TQ]
                   + prod[3 * Cout:4 * Cout, WSP + 1:WSP + TQ + 1])
            bn = acc + b_ref[...]
            pos = jnp.maximum(bn, 0.0)
            neg = pos - bn
            ob = jnp.concatenate([pos, neg], axis=0).astype(o_ref.dtype)
            o_ref[...] = ob.reshape(2 * Cout, T, WSP)[:, :, :OW]

    cost = pl.CostEstimate(
        flops=2 * N * R * M * CE * L,
        transcendentals=0,
        bytes_accessed=(x.size * 4 + w_all.size * 2 + bias.size * 4
                        + N * 2 * Cout * OH * OW * 2),
    )

    out = pl.pallas_call(
        body,
        out_shape=jax.ShapeDtypeStruct((N, 2 * Cout, OH, OW), jnp.bfloat16),
        grid=(N, R + 1),
        in_specs=[
            pl.BlockSpec((None, C, HB, W),
                         lambda n, r: (n, 0, jnp.minimum(r, R - 1), 0)),
            pl.BlockSpec((M, CE), lambda n, r: (0, 0)),
            pl.BlockSpec((Cout, 1), lambda n, r: (0, 0)),
        ],
        out_specs=pl.BlockSpec((None, 2 * Cout, T, OW),
                               lambda n, r: (n, 0, jnp.maximum(r - 1, 0), 0)),
        scratch_shapes=[pltpu.VMEM((CE, Hs, WSP), jnp.bfloat16),
                        pltpu.VMEM((C, W, CHK), jnp.float32),
                        pltpu.VMEM((C, s, NP, HB, 128), jnp.float32)],
        compiler_params=pltpu.CompilerParams(
            dimension_semantics=("parallel", "arbitrary"),
            vmem_limit_bytes=100 * 1024 * 1024),
        cost_estimate=cost,
    )(x, w_all, bias)
    return out


@jax.jit
def kernel(x, weight, gamma, beta, running_mean, running_var):
    return _crelu(x, weight, gamma, beta, running_mean, running_var,
                  stride=4, padding=3)


# speedup vs baseline: 1.1239x; 1.0204x over previous
import jax
import jax.numpy as jnp
from jax.experimental import pallas as pl
from jax.experimental.pallas import tpu as pltpu


def _crelu(x, weight, gamma, beta, running_mean, running_var,
           stride=4, padding=3, eps=1e-5):
    N, C, H, W = x.shape
    Cout, Cin, KH, KW = weight.shape
    s = int(stride)
    p = int(padding)
    assert s == 4 and p == 3 and Cin == C and KH == 7 and KW == 7
    OH = (H + 2 * p - KH) // s + 1
    OW = (W + 2 * p - KW) // s + 1
    KHs = 2
    KWs = 2
    CE = s * s * C

    T = next(t for t in (64, 32, 16, 8, 4, 2, 1) if OH % t == 0)
    R = OH // T
    HB = H // R
    Hs = OH + KHs
    WSP = (OW + KWs + 127) // 128 * 128
    L = (T + KHs) * WSP
    TQ = T * WSP
    NP = -(-OW // 128)
    CHK = min(128, HB)

    scale = gamma.astype(jnp.float32) * jax.lax.rsqrt(
        running_var.astype(jnp.float32) + eps)
    bias = (beta.astype(jnp.float32)
            - running_mean.astype(jnp.float32) * scale).reshape(Cout, 1)
    w_f = weight.astype(jnp.float32) * scale[:, None, None, None]

    w_pad = jnp.pad(w_f, ((0, 0), (0, 0), (0, KHs * s - KH), (0, KWs * s - KW)))
    wg = w_pad.reshape(Cout, C, KHs, s, KWs, s)
    wg = wg.transpose(2, 4, 0, 3, 5, 1)
    w_all = wg.reshape(KHs * KWs * Cout, CE).astype(jnp.bfloat16)

    M = KHs * KWs * Cout

    NR = N * R
    FLAT = R >= 3

    def body(x_ref, w_ref, b_ref, o_ref, xs3, xt_s, xw_s):
        if FLAT:
            st = pl.program_id(0)
            k = st % R
            zero_cond = st == 0
            build_cond = st < NR
            compute_cond = st > 0
            kt = (st - 1) % R
        else:
            k = pl.program_id(1)
            zero_cond = k == 0
            build_cond = k < R
            compute_cond = k > 0
            kt = k - 1

        @pl.when(zero_cond)
        def _zero():
            z_r = jnp.zeros((CE, 1, WSP), jnp.bfloat16)
            xs3[:, 0:1, :] = z_r
            xs3[:, OH:OH + KHs, :] = jnp.zeros((CE, KHs, WSP), jnp.bfloat16)
            z_c = jnp.zeros((C * (s - 1), Hs, 1), jnp.bfloat16)
            z_c3 = jnp.zeros((C, Hs, 1), jnp.bfloat16)
            for rt in range(s):
                xs3[rt * s * C:(rt * s + 3) * C, :, 0:1] = z_c
                xs3[(rt * s + 3) * C:(rt + 1) * s * C, :, OW:OW + 1] = z_c3

        @pl.when(build_cond)
        def _build():
            for kc in range(HB // CHK):
                rows = x_ref[:, kc * CHK:(kc + 1) * CHK, :]
                xt_s[...] = jnp.swapaxes(rows, 1, 2)
                for t in range(s):
                    wp0 = 0 if t == 3 else 1
                    cs0 = 4 * wp0 + t - 3
                    colsT = xt_s[:, cs0:cs0 + 4 * OW:4, :]
                    cols = jnp.swapaxes(colsT, 1, 2)
                    for i in range(NP):
                        pw = min(128, OW - 128 * i)
                        xw_s[:, t, i, kc * CHK:(kc + 1) * CHK, 0:pw] = (
                            cols[:, :, 128 * i:128 * i + pw])
            for kk in range(R):
                @pl.when(k == kk)
                def _store(kk=kk):
                    for rt in range(s):
                        hs0 = 0 if rt == 3 else 1
                        rs0 = 4 * hs0 + rt - 3
                        for t in range(s):
                            wp0 = 0 if t == 3 else 1
                            ce0 = (rt * s + t) * C
                            part = xw_s[:, t, :, rs0:rs0 + 4 * T:4,
                                        :].astype(jnp.bfloat16)
                            for i in range(NP):
                                pw = min(128, OW - 128 * i)
                                xs3[ce0:ce0 + C,
                                    hs0 + T * kk:hs0 + T * kk + T,
                                    wp0 + 128 * i:wp0 + 128 * i + pw] = (
                                    part[:, i, :, 0:pw])

        @pl.when(compute_cond)
        def _compute():
            xf = xs3[:, pl.ds(T * kt, T + KHs), :].reshape(CE, L)
            prod = jnp.dot(w_ref[...], xf,
                           preferred_element_type=jnp.float32)
            acc = (prod[0 * Cout:1 * Cout, 0:TQ]
                   + prod[1 * Cout:2 * Cout, 1:TQ + 1]
                   + prod[2 * Cout:3 * Cout, WSP:WSP + TQ]
                   + prod[3 * Cout:4 * Cout, WSP + 1:WSP + TQ + 1])
            bn = acc + b_ref[...]
            pos = jnp.maximum(bn, 0.0)
            neg = pos - bn
            ob = jnp.concatenate([pos, neg], axis=0).astype(o_ref.dtype)
            o_ref[...] = ob.reshape(2 * Cout, T, WSP)[:, :, :OW]

    cost = pl.CostEstimate(
        flops=2 * N * R * M * CE * L,
        transcendentals=0,
        bytes_accessed=(x.size * 4 + w_all.size * 2 + bias.size * 4
                        + N * 2 * Cout * OH * OW * 2),
    )

    out = pl.pallas_call(
        body,
        out_shape=jax.ShapeDtypeStruct((N, 2 * Cout, OH, OW), jnp.bfloat16),
        grid=(N * R + 1,) if FLAT else (N, R + 1),
        in_specs=[
            pl.BlockSpec(
                (None, C, HB, W),
                (lambda st: (jnp.minimum(st, N * R - 1) // R, 0,
                             jnp.minimum(st, N * R - 1) % R, 0)) if FLAT else
                (lambda n, r: (n, 0, jnp.minimum(r, R - 1), 0))),
            pl.BlockSpec((M, CE),
                         (lambda st: (0, 0)) if FLAT else
                         (lambda n, r: (0, 0))),
            pl.BlockSpec((Cout, 1),
                         (lambda st: (0, 0)) if FLAT else
                         (lambda n, r: (0, 0))),
        ],
        out_specs=pl.BlockSpec(
            (None, 2 * Cout, T, OW),
            (lambda st: (jnp.maximum(st - 1, 0) // R, 0,
                         jnp.maximum(st - 1, 0) % R, 0)) if FLAT else
            (lambda n, r: (n, 0, jnp.maximum(r - 1, 0), 0))),
        scratch_shapes=[pltpu.VMEM((CE, Hs, WSP), jnp.bfloat16),
                        pltpu.VMEM((C, W, CHK), jnp.float32),
                        pltpu.VMEM((C, s, NP, HB, 128), jnp.float32)],
        compiler_params=pltpu.CompilerParams(
            dimension_semantics=("arbitrary",) if FLAT
            else ("arbitrary", "arbitrary"),
            vmem_limit_bytes=100 * 1024 * 1024),
        cost_estimate=cost,
    )(x, w_all, bias)
    return out


@jax.jit
def kernel(x, weight, gamma, beta, running_mean, running_var):
    return _crelu(x, weight, gamma, beta, running_mean, running_var,
                  stride=4, padding=3)
